# baseline (speedup 1.0000x reference)
import functools

import jax
import jax.numpy as jnp
from jax import lax
from jax.experimental import pallas as pl
from jax.experimental.pallas import tpu as pltpu
from jax.experimental.pallas import tpu_sc as plsc

D_ = 1024
E_ = 8
H_ = 2048
T_ = 2048
CHUNK_ = 256
TM_ = 128
NP_ = T_ * 2 // TM_ + E_
P_ = NP_ * TM_
_NW = 32
_RB = T_ // _NW


def _router_body(x_ref, wr_ref, d1_ref, d2_ref, wt1_ref, wt2_ref, cnt_ref):
    xb16 = x_ref[...].astype(jnp.bfloat16)
    logits = lax.dot_general(
        xb16, wr_ref[...].astype(jnp.bfloat16), (((1,), (0,)), ((), ())),
        preferred_element_type=jnp.float32)
    mx = jnp.max(logits, axis=1, keepdims=True)
    ex = jnp.exp(logits - mx)
    p = ex / jnp.sum(ex, axis=1, keepdims=True)
    idx = lax.broadcasted_iota(jnp.int32, p.shape, 1)
    big = jnp.int32(E_)
    m1 = jnp.max(p, axis=1, keepdims=True)
    i1 = jnp.min(jnp.where(p == m1, idx, big), axis=1, keepdims=True)
    p2 = jnp.where(idx == i1, -jnp.inf, p)
    m2 = jnp.max(p2, axis=1, keepdims=True)
    i2 = jnp.min(jnp.where(p2 == m2, idx, big), axis=1, keepdims=True)
    wt1_ref[...] = m1
    wt2_ref[...] = m2

    sel = jnp.logical_or(idx == i1, idx == i2).astype(jnp.bfloat16)

    r_io = lax.broadcasted_iota(jnp.int32, (CHUNK_, CHUNK_), 0)
    c_io = lax.broadcasted_iota(jnp.int32, (CHUNK_, CHUNK_), 1)
    lt = (c_io < r_io).astype(jnp.bfloat16)
    carry = jnp.zeros((1, E_), jnp.float32)
    ranks = []
    for c in range(T_ // CHUNK_):
        sel_c = sel[c * CHUNK_:(c + 1) * CHUNK_, :]
        rk = lax.dot_general(lt, sel_c, (((1,), (0,)), ((), ())),
                             preferred_element_type=jnp.float32)
        ranks.append(rk + carry)
        carry = carry + jnp.sum(sel_c, axis=0, keepdims=True).astype(jnp.float32)

    counts = carry
    ntp = jnp.floor((counts + (TM_ - 1)) * (1.0 / TM_)).astype(jnp.float32) * TM_
    u_r = lax.broadcasted_iota(jnp.int32, (E_, E_), 0)
    u_c = lax.broadcasted_iota(jnp.int32, (E_, E_), 1)
    ut = (u_r < u_c).astype(jnp.float32)
    po = lax.dot_general(ntp, ut, (((1,), (0,)), ((), ())),
                         precision=lax.Precision.HIGHEST,
                         preferred_element_type=jnp.float32)
    cnt_ref[...] = counts.astype(jnp.int32)

    for c in range(T_ // CHUNK_):
        rows = pl.ds(c * CHUNK_, CHUNK_)
        rk = ranks[c]
        i1c = i1[c * CHUNK_:(c + 1) * CHUNK_, :]
        i2c = i2[c * CHUNK_:(c + 1) * CHUNK_, :]
        idx_c = idx[:CHUNK_, :]
        r1 = jnp.sum(jnp.where(idx_c == i1c, rk, 0.0), axis=1, keepdims=True)
        r2 = jnp.sum(jnp.where(idx_c == i2c, rk, 0.0), axis=1, keepdims=True)
        o1 = jnp.sum(jnp.where(idx_c == i1c, po, 0.0), axis=1, keepdims=True)
        o2 = jnp.sum(jnp.where(idx_c == i2c, po, 0.0), axis=1, keepdims=True)
        d1_ref[rows, :] = (r1 + o1).astype(jnp.int32)
        d2_ref[rows, :] = (r2 + o2).astype(jnp.int32)


def _router(x, w_router):
    return pl.pallas_call(
        _router_body,
        in_specs=[
            pl.BlockSpec((T_, D_), lambda: (0, 0)),
            pl.BlockSpec((D_, E_), lambda: (0, 0)),
        ],
        out_specs=[
            pl.BlockSpec((T_, 1), lambda: (0, 0)),
            pl.BlockSpec((T_, 1), lambda: (0, 0)),
            pl.BlockSpec((T_, 1), lambda: (0, 0)),
            pl.BlockSpec((T_, 1), lambda: (0, 0)),
            pl.BlockSpec((1, E_), lambda: (0, 0)),
        ],
        out_shape=[
            jax.ShapeDtypeStruct((T_, 1), jnp.int32),
            jax.ShapeDtypeStruct((T_, 1), jnp.int32),
            jax.ShapeDtypeStruct((T_, 1), jnp.float32),
            jax.ShapeDtypeStruct((T_, 1), jnp.float32),
            jax.ShapeDtypeStruct((1, E_), jnp.int32),
        ],
    )(x, w_router)


def _make_scatter():
    mesh = plsc.VectorSubcoreMesh(core_axis_name="core",
                                  subcore_axis_name="subcore")

    @functools.partial(
        pl.kernel, mesh=mesh,
        out_type=jax.ShapeDtypeStruct((P_, D_), jnp.float32),
        scratch_types=[pltpu.VMEM((_RB, D_), jnp.float32),
                       pltpu.VMEM((_RB,), jnp.int32),
                       pltpu.VMEM((_RB,), jnp.int32),
                       pltpu.SemaphoreType.DMA])
    def scatter_kernel(x_hbm, d1_hbm, d2_hbm, xs_hbm, xrows, i1v, i2v, sem):
        wid = (lax.axis_index("subcore") * plsc.get_sparse_core_info().num_cores
               + lax.axis_index("core"))
        base = wid * _RB
        pltpu.sync_copy(x_hbm.at[pl.ds(base, _RB)], xrows)
        pltpu.sync_copy(d1_hbm.at[pl.ds(base, _RB)], i1v)
        pltpu.sync_copy(d2_hbm.at[pl.ds(base, _RB)], i2v)
        c1 = pltpu.async_copy(xrows, xs_hbm.at[i1v], sem)
        c2 = pltpu.async_copy(xrows, xs_hbm.at[i2v], sem)
        c1.wait()
        c2.wait()

    return scatter_kernel


def _grouped_body(te_ref, used_ref, xs_ref, w1_ref, w2_ref, ys_ref,
                  w1c_ref, w2c_ref):
    i = pl.program_id(0)
    prev = te_ref[jnp.maximum(i - 1, 0)]
    changed = jnp.logical_or(i == 0, te_ref[i] != prev)

    @pl.when(changed)
    def _cvt():
        w1c_ref[...] = w1_ref[0].astype(jnp.bfloat16)
        w2c_ref[...] = w2_ref[0].astype(jnp.bfloat16)

    @pl.when(used_ref[i] != 0)
    def _compute():
        xb = xs_ref[...].astype(jnp.bfloat16)
        h = lax.dot_general(xb, w1c_ref[...], (((1,), (0,)), ((), ())),
                            preferred_element_type=jnp.float32)
        h = h * jax.nn.sigmoid(h)
        ys_ref[...] = lax.dot_general(h.astype(jnp.bfloat16), w2c_ref[...],
                                      (((1,), (0,)), ((), ())),
                                      preferred_element_type=jnp.float32)


def _grouped(xs, w1, w2, tile_expert, used):
    return pl.pallas_call(
        _grouped_body,
        grid_spec=pltpu.PrefetchScalarGridSpec(
            num_scalar_prefetch=2,
            grid=(NP_,),
            in_specs=[
                pl.BlockSpec((TM_, D_), lambda i, te, u: (i, 0)),
                pl.BlockSpec((1, D_, H_), lambda i, te, u: (te[i], 0, 0)),
                pl.BlockSpec((1, H_, D_), lambda i, te, u: (te[i], 0, 0)),
            ],
            out_specs=pl.BlockSpec((TM_, D_), lambda i, te, u: (i, 0)),
            scratch_shapes=[
                pltpu.VMEM((D_, H_), jnp.bfloat16),
                pltpu.VMEM((H_, D_), jnp.bfloat16),
            ],
        ),
        out_shape=jax.ShapeDtypeStruct((P_, D_), jnp.float32),
        compiler_params=pltpu.CompilerParams(
            dimension_semantics=("arbitrary",),
        ),
    )(tile_expert, used, xs, w1, w2)


def _make_gather():
    mesh = plsc.VectorSubcoreMesh(core_axis_name="core",
                                  subcore_axis_name="subcore")

    sb = _RB // 2

    @functools.partial(
        pl.kernel, mesh=mesh,
        out_type=[jax.ShapeDtypeStruct((T_, D_), jnp.float32),
                  jax.ShapeDtypeStruct((T_, D_), jnp.float32)],
        scratch_types=[pltpu.VMEM((sb, D_), jnp.float32),
                       pltpu.VMEM((sb, D_), jnp.float32),
                       pltpu.VMEM((sb,), jnp.int32),
                       pltpu.VMEM((sb,), jnp.int32),
                       pltpu.SemaphoreType.DMA])
    def gather_kernel(ys_hbm, d1_hbm, d2_hbm, g1_hbm, g2_hbm,
                      b1, b2, i1v, i2v, sem):
        wid = (lax.axis_index("subcore") * plsc.get_sparse_core_info().num_cores
               + lax.axis_index("core"))
        for s in range(2):
            base = wid * _RB + s * sb
            pltpu.sync_copy(d1_hbm.at[pl.ds(base, sb)], i1v)
            pltpu.sync_copy(d2_hbm.at[pl.ds(base, sb)], i2v)
            c1 = pltpu.async_copy(ys_hbm.at[i1v], b1, sem)
            c2 = pltpu.async_copy(ys_hbm.at[i2v], b2, sem)
            c1.wait()
            c2.wait()
            pltpu.sync_copy(b1, g1_hbm.at[pl.ds(base, sb)])
            pltpu.sync_copy(b2, g2_hbm.at[pl.ds(base, sb)])

    return gather_kernel


def _combine_body(g1_ref, g2_ref, wt1_ref, wt2_ref, out_ref):
    out_ref[...] = wt1_ref[...] * g1_ref[...] + wt2_ref[...] * g2_ref[...]


def _combine(g1, g2, wt1, wt2):
    nblk = T_ // CHUNK_
    return pl.pallas_call(
        _combine_body,
        grid=(nblk,),
        in_specs=[
            pl.BlockSpec((CHUNK_, D_), lambda i: (i, 0)),
            pl.BlockSpec((CHUNK_, D_), lambda i: (i, 0)),
            pl.BlockSpec((CHUNK_, 1), lambda i: (i, 0)),
            pl.BlockSpec((CHUNK_, 1), lambda i: (i, 0)),
        ],
        out_specs=pl.BlockSpec((CHUNK_, D_), lambda i: (i, 0)),
        out_shape=jax.ShapeDtypeStruct((T_, D_), jnp.float32),
    )(g1, g2, wt1, wt2)


@jax.jit
def kernel(x, w_router, w1, w2):
    d1, d2, wt1, wt2, counts = _router(x, w_router)

    c = counts[0]
    nt = (c + (TM_ - 1)) // TM_
    cumtiles = jnp.cumsum(nt)
    tidx = jnp.arange(NP_, dtype=jnp.int32)
    tile_expert = jnp.minimum(
        jnp.sum(tidx[:, None] >= cumtiles[None, :], axis=1).astype(jnp.int32),
        E_ - 1)
    seg_end = ((cumtiles - nt) * TM_ + c)[tile_expert]
    used = (tidx * TM_ < seg_end).astype(jnp.int32)

    d1r = d1.reshape(T_)
    d2r = d2.reshape(T_)

    xs = _make_scatter()(x, d1r, d2r)
    ys = _grouped(xs, w1, w2, tile_expert, used)
    g1, g2 = _make_gather()(ys, d1r, d2r)
    return _combine(g1, g2, wt1, wt2)


# speedup vs baseline: 1.0148x; 1.0148x over previous
import functools

import jax
import jax.numpy as jnp
from jax import lax
from jax.experimental import pallas as pl
from jax.experimental.pallas import tpu as pltpu
from jax.experimental.pallas import tpu_sc as plsc

D_ = 1024
E_ = 8
H_ = 2048
T_ = 2048
CHUNK_ = 256
TM_ = 256
NP_ = T_ * 2 // TM_ + E_
P_ = NP_ * TM_
_NW = 32
_RB = T_ // _NW


def _router_body(x_ref, wr_ref, d1_ref, d2_ref, wt1_ref, wt2_ref, cnt_ref):
    xb16 = x_ref[...].astype(jnp.bfloat16)
    logits = lax.dot_general(
        xb16, wr_ref[...].astype(jnp.bfloat16), (((1,), (0,)), ((), ())),
        preferred_element_type=jnp.float32)
    mx = jnp.max(logits, axis=1, keepdims=True)
    ex = jnp.exp(logits - mx)
    p = ex / jnp.sum(ex, axis=1, keepdims=True)
    idx = lax.broadcasted_iota(jnp.int32, p.shape, 1)
    big = jnp.int32(E_)
    m1 = jnp.max(p, axis=1, keepdims=True)
    i1 = jnp.min(jnp.where(p == m1, idx, big), axis=1, keepdims=True)
    p2 = jnp.where(idx == i1, -jnp.inf, p)
    m2 = jnp.max(p2, axis=1, keepdims=True)
    i2 = jnp.min(jnp.where(p2 == m2, idx, big), axis=1, keepdims=True)
    wt1_ref[...] = m1
    wt2_ref[...] = m2

    sel = jnp.logical_or(idx == i1, idx == i2).astype(jnp.bfloat16)

    r_io = lax.broadcasted_iota(jnp.int32, (CHUNK_, CHUNK_), 0)
    c_io = lax.broadcasted_iota(jnp.int32, (CHUNK_, CHUNK_), 1)
    lt = (c_io < r_io).astype(jnp.bfloat16)
    carry = jnp.zeros((1, E_), jnp.float32)
    ranks = []
    for c in range(T_ // CHUNK_):
        sel_c = sel[c * CHUNK_:(c + 1) * CHUNK_, :]
        rk = lax.dot_general(lt, sel_c, (((1,), (0,)), ((), ())),
                             preferred_element_type=jnp.float32)
        ranks.append(rk + carry)
        carry = carry + jnp.sum(sel_c, axis=0, keepdims=True).astype(jnp.float32)

    counts = carry
    ntp = jnp.floor((counts + (TM_ - 1)) * (1.0 / TM_)).astype(jnp.float32) * TM_
    u_r = lax.broadcasted_iota(jnp.int32, (E_, E_), 0)
    u_c = lax.broadcasted_iota(jnp.int32, (E_, E_), 1)
    ut = (u_r < u_c).astype(jnp.float32)
    po = lax.dot_general(ntp, ut, (((1,), (0,)), ((), ())),
                         precision=lax.Precision.HIGHEST,
                         preferred_element_type=jnp.float32)
    cnt_ref[...] = counts.astype(jnp.int32)

    for c in range(T_ // CHUNK_):
        rows = pl.ds(c * CHUNK_, CHUNK_)
        rk = ranks[c]
        i1c = i1[c * CHUNK_:(c + 1) * CHUNK_, :]
        i2c = i2[c * CHUNK_:(c + 1) * CHUNK_, :]
        idx_c = idx[:CHUNK_, :]
        r1 = jnp.sum(jnp.where(idx_c == i1c, rk, 0.0), axis=1, keepdims=True)
        r2 = jnp.sum(jnp.where(idx_c == i2c, rk, 0.0), axis=1, keepdims=True)
        o1 = jnp.sum(jnp.where(idx_c == i1c, po, 0.0), axis=1, keepdims=True)
        o2 = jnp.sum(jnp.where(idx_c == i2c, po, 0.0), axis=1, keepdims=True)
        d1_ref[rows, :] = (r1 + o1).astype(jnp.int32)
        d2_ref[rows, :] = (r2 + o2).astype(jnp.int32)


def _router(x, w_router):
    return pl.pallas_call(
        _router_body,
        in_specs=[
            pl.BlockSpec((T_, D_), lambda: (0, 0)),
            pl.BlockSpec((D_, E_), lambda: (0, 0)),
        ],
        out_specs=[
            pl.BlockSpec((T_, 1), lambda: (0, 0)),
            pl.BlockSpec((T_, 1), lambda: (0, 0)),
            pl.BlockSpec((T_, 1), lambda: (0, 0)),
            pl.BlockSpec((T_, 1), lambda: (0, 0)),
            pl.BlockSpec((1, E_), lambda: (0, 0)),
        ],
        out_shape=[
            jax.ShapeDtypeStruct((T_, 1), jnp.int32),
            jax.ShapeDtypeStruct((T_, 1), jnp.int32),
            jax.ShapeDtypeStruct((T_, 1), jnp.float32),
            jax.ShapeDtypeStruct((T_, 1), jnp.float32),
            jax.ShapeDtypeStruct((1, E_), jnp.int32),
        ],
    )(x, w_router)


def _make_scatter():
    mesh = plsc.VectorSubcoreMesh(core_axis_name="core",
                                  subcore_axis_name="subcore")

    @functools.partial(
        pl.kernel, mesh=mesh,
        out_type=jax.ShapeDtypeStruct((P_, D_), jnp.float32),
        scratch_types=[pltpu.VMEM((_RB, D_), jnp.float32),
                       pltpu.VMEM((_RB,), jnp.int32),
                       pltpu.VMEM((_RB,), jnp.int32),
                       pltpu.SemaphoreType.DMA])
    def scatter_kernel(x_hbm, d1_hbm, d2_hbm, xs_hbm, xrows, i1v, i2v, sem):
        wid = (lax.axis_index("subcore") * plsc.get_sparse_core_info().num_cores
               + lax.axis_index("core"))
        base = wid * _RB
        pltpu.sync_copy(x_hbm.at[pl.ds(base, _RB)], xrows)
        pltpu.sync_copy(d1_hbm.at[pl.ds(base, _RB)], i1v)
        pltpu.sync_copy(d2_hbm.at[pl.ds(base, _RB)], i2v)
        c1 = pltpu.async_copy(xrows, xs_hbm.at[i1v], sem)
        c2 = pltpu.async_copy(xrows, xs_hbm.at[i2v], sem)
        c1.wait()
        c2.wait()

    return scatter_kernel


def _grouped_body(te_ref, used_ref, xs_ref, w1_ref, w2_ref, ys_ref,
                  w1c_ref, w2c_ref, h_ref):
    i = pl.program_id(0)
    prev = te_ref[jnp.maximum(i - 1, 0)]
    changed = jnp.logical_or(i == 0, te_ref[i] != prev)
    used = used_ref[i] != 0

    @pl.when(changed)
    def _cvt1():
        w1c_ref[...] = w1_ref[0].astype(jnp.bfloat16)

    @pl.when(used)
    def _mm1():
        xb = xs_ref[...].astype(jnp.bfloat16)
        h = lax.dot_general(xb, w1c_ref[...], (((1,), (0,)), ((), ())),
                            preferred_element_type=jnp.float32)
        h = h * jax.nn.sigmoid(h)
        h_ref[...] = h.astype(jnp.bfloat16)

    @pl.when(changed)
    def _cvt2():
        w2c_ref[...] = w2_ref[0].astype(jnp.bfloat16)

    @pl.when(used)
    def _mm2():
        ys_ref[...] = lax.dot_general(h_ref[...], w2c_ref[...],
                                      (((1,), (0,)), ((), ())),
                                      preferred_element_type=jnp.float32)


def _grouped(xs, w1, w2, tile_expert, used):
    return pl.pallas_call(
        _grouped_body,
        grid_spec=pltpu.PrefetchScalarGridSpec(
            num_scalar_prefetch=2,
            grid=(NP_,),
            in_specs=[
                pl.BlockSpec((TM_, D_), lambda i, te, u: (i, 0)),
                pl.BlockSpec((1, D_, H_), lambda i, te, u: (te[i], 0, 0)),
                pl.BlockSpec((1, H_, D_), lambda i, te, u: (te[i], 0, 0)),
            ],
            out_specs=pl.BlockSpec((TM_, D_), lambda i, te, u: (i, 0)),
            scratch_shapes=[
                pltpu.VMEM((D_, H_), jnp.bfloat16),
                pltpu.VMEM((H_, D_), jnp.bfloat16),
                pltpu.VMEM((TM_, H_), jnp.bfloat16),
            ],
        ),
        out_shape=jax.ShapeDtypeStruct((P_, D_), jnp.float32),
        compiler_params=pltpu.CompilerParams(
            dimension_semantics=("arbitrary",),
        ),
    )(tile_expert, used, xs, w1, w2)


def _make_gather():
    mesh = plsc.VectorSubcoreMesh(core_axis_name="core",
                                  subcore_axis_name="subcore")

    sb = _RB // 2

    @functools.partial(
        pl.kernel, mesh=mesh,
        out_type=[jax.ShapeDtypeStruct((T_, D_), jnp.float32),
                  jax.ShapeDtypeStruct((T_, D_), jnp.float32)],
        scratch_types=[pltpu.VMEM((sb, D_), jnp.float32),
                       pltpu.VMEM((sb, D_), jnp.float32),
                       pltpu.VMEM((sb,), jnp.int32),
                       pltpu.VMEM((sb,), jnp.int32),
                       pltpu.SemaphoreType.DMA])
    def gather_kernel(ys_hbm, d1_hbm, d2_hbm, g1_hbm, g2_hbm,
                      b1, b2, i1v, i2v, sem):
        wid = (lax.axis_index("subcore") * plsc.get_sparse_core_info().num_cores
               + lax.axis_index("core"))
        for s in range(2):
            base = wid * _RB + s * sb
            pltpu.sync_copy(d1_hbm.at[pl.ds(base, sb)], i1v)
            pltpu.sync_copy(d2_hbm.at[pl.ds(base, sb)], i2v)
            c1 = pltpu.async_copy(ys_hbm.at[i1v], b1, sem)
            c2 = pltpu.async_copy(ys_hbm.at[i2v], b2, sem)
            c1.wait()
            c2.wait()
            pltpu.sync_copy(b1, g1_hbm.at[pl.ds(base, sb)])
            pltpu.sync_copy(b2, g2_hbm.at[pl.ds(base, sb)])

    return gather_kernel


def _combine_body(g1_ref, g2_ref, wt1_ref, wt2_ref, out_ref):
    out_ref[...] = wt1_ref[...] * g1_ref[...] + wt2_ref[...] * g2_ref[...]


def _combine(g1, g2, wt1, wt2):
    nblk = T_ // CHUNK_
    return pl.pallas_call(
        _combine_body,
        grid=(nblk,),
        in_specs=[
            pl.BlockSpec((CHUNK_, D_), lambda i: (i, 0)),
            pl.BlockSpec((CHUNK_, D_), lambda i: (i, 0)),
            pl.BlockSpec((CHUNK_, 1), lambda i: (i, 0)),
            pl.BlockSpec((CHUNK_, 1), lambda i: (i, 0)),
        ],
        out_specs=pl.BlockSpec((CHUNK_, D_), lambda i: (i, 0)),
        out_shape=jax.ShapeDtypeStruct((T_, D_), jnp.float32),
    )(g1, g2, wt1, wt2)


@jax.jit
def kernel(x, w_router, w1, w2):
    d1, d2, wt1, wt2, counts = _router(x, w_router)

    c = counts[0]
    nt = (c + (TM_ - 1)) // TM_
    cumtiles = jnp.cumsum(nt)
    tidx = jnp.arange(NP_, dtype=jnp.int32)
    tile_expert = jnp.minimum(
        jnp.sum(tidx[:, None] >= cumtiles[None, :], axis=1).astype(jnp.int32),
        E_ - 1)
    seg_end = ((cumtiles - nt) * TM_ + c)[tile_expert]
    used = (tidx * TM_ < seg_end).astype(jnp.int32)

    d1r = d1.reshape(T_)
    d2r = d2.reshape(T_)

    xs = _make_scatter()(x, d1r, d2r)
    ys = _grouped(xs, w1, w2, tile_expert, used)
    g1, g2 = _make_gather()(ys, d1r, d2r)
    return _combine(g1, g2, wt1, wt2)


# speedup vs baseline: 1.0391x; 1.0240x over previous
import functools

import jax
import jax.numpy as jnp
from jax import lax
from jax.experimental import pallas as pl
from jax.experimental.pallas import tpu as pltpu
from jax.experimental.pallas import tpu_sc as plsc

D_ = 1024
E_ = 8
H_ = 2048
T_ = 2048
CHUNK_ = 256
TM_ = 256
NP_ = T_ * 2 // TM_ + E_
P_ = NP_ * TM_
_NW = 32
_RB = T_ // _NW


def _router_body(x_ref, wr_ref, d1_ref, d2_ref, wt1_ref, wt2_ref, cnt_ref):
    xb16 = x_ref[...].astype(jnp.bfloat16)
    logits = lax.dot_general(
        xb16, wr_ref[...].astype(jnp.bfloat16), (((1,), (0,)), ((), ())),
        preferred_element_type=jnp.float32)
    mx = jnp.max(logits, axis=1, keepdims=True)
    ex = jnp.exp(logits - mx)
    p = ex / jnp.sum(ex, axis=1, keepdims=True)
    idx = lax.broadcasted_iota(jnp.int32, p.shape, 1)
    big = jnp.int32(E_)
    m1 = jnp.max(p, axis=1, keepdims=True)
    i1 = jnp.min(jnp.where(p == m1, idx, big), axis=1, keepdims=True)
    p2 = jnp.where(idx == i1, -jnp.inf, p)
    m2 = jnp.max(p2, axis=1, keepdims=True)
    i2 = jnp.min(jnp.where(p2 == m2, idx, big), axis=1, keepdims=True)
    wt1_ref[...] = m1
    wt2_ref[...] = m2

    sel = jnp.logical_or(idx == i1, idx == i2).astype(jnp.bfloat16)

    r_io = lax.broadcasted_iota(jnp.int32, (CHUNK_, CHUNK_), 0)
    c_io = lax.broadcasted_iota(jnp.int32, (CHUNK_, CHUNK_), 1)
    lt = (c_io < r_io).astype(jnp.bfloat16)
    carry = jnp.zeros((1, E_), jnp.float32)
    ranks = []
    for c in range(T_ // CHUNK_):
        sel_c = sel[c * CHUNK_:(c + 1) * CHUNK_, :]
        rk = lax.dot_general(lt, sel_c, (((1,), (0,)), ((), ())),
                             preferred_element_type=jnp.float32)
        ranks.append(rk + carry)
        carry = carry + jnp.sum(sel_c, axis=0, keepdims=True).astype(jnp.float32)

    counts = carry
    ntp = jnp.floor((counts + (TM_ - 1)) * (1.0 / TM_)).astype(jnp.float32) * TM_
    u_r = lax.broadcasted_iota(jnp.int32, (E_, E_), 0)
    u_c = lax.broadcasted_iota(jnp.int32, (E_, E_), 1)
    ut = (u_r < u_c).astype(jnp.float32)
    po = lax.dot_general(ntp, ut, (((1,), (0,)), ((), ())),
                         precision=lax.Precision.HIGHEST,
                         preferred_element_type=jnp.float32)
    cnt_ref[...] = counts.astype(jnp.int32)

    for c in range(T_ // CHUNK_):
        rows = pl.ds(c * CHUNK_, CHUNK_)
        rk = ranks[c]
        i1c = i1[c * CHUNK_:(c + 1) * CHUNK_, :]
        i2c = i2[c * CHUNK_:(c + 1) * CHUNK_, :]
        idx_c = idx[:CHUNK_, :]
        r1 = jnp.sum(jnp.where(idx_c == i1c, rk, 0.0), axis=1, keepdims=True)
        r2 = jnp.sum(jnp.where(idx_c == i2c, rk, 0.0), axis=1, keepdims=True)
        o1 = jnp.sum(jnp.where(idx_c == i1c, po, 0.0), axis=1, keepdims=True)
        o2 = jnp.sum(jnp.where(idx_c == i2c, po, 0.0), axis=1, keepdims=True)
        d1_ref[rows, :] = (r1 + o1).astype(jnp.int32)
        d2_ref[rows, :] = (r2 + o2).astype(jnp.int32)


def _router(x, w_router):
    return pl.pallas_call(
        _router_body,
        in_specs=[
            pl.BlockSpec((T_, D_), lambda: (0, 0)),
            pl.BlockSpec((D_, E_), lambda: (0, 0)),
        ],
        out_specs=[
            pl.BlockSpec((T_, 1), lambda: (0, 0)),
            pl.BlockSpec((T_, 1), lambda: (0, 0)),
            pl.BlockSpec((T_, 1), lambda: (0, 0)),
            pl.BlockSpec((T_, 1), lambda: (0, 0)),
            pl.BlockSpec((1, E_), lambda: (0, 0)),
        ],
        out_shape=[
            jax.ShapeDtypeStruct((T_, 1), jnp.int32),
            jax.ShapeDtypeStruct((T_, 1), jnp.int32),
            jax.ShapeDtypeStruct((T_, 1), jnp.float32),
            jax.ShapeDtypeStruct((T_, 1), jnp.float32),
            jax.ShapeDtypeStruct((1, E_), jnp.int32),
        ],
    )(x, w_router)


def _make_scatter():
    mesh = plsc.VectorSubcoreMesh(core_axis_name="core",
                                  subcore_axis_name="subcore")

    @functools.partial(
        pl.kernel, mesh=mesh,
        out_type=jax.ShapeDtypeStruct((P_, D_), jnp.float32),
        scratch_types=[pltpu.VMEM((_RB, D_), jnp.float32),
                       pltpu.VMEM((_RB,), jnp.int32),
                       pltpu.VMEM((_RB,), jnp.int32),
                       pltpu.SemaphoreType.DMA])
    def scatter_kernel(x_hbm, d1_hbm, d2_hbm, xs_hbm, xrows, i1v, i2v, sem):
        wid = (lax.axis_index("subcore") * plsc.get_sparse_core_info().num_cores
               + lax.axis_index("core"))
        base = wid * _RB
        pltpu.sync_copy(x_hbm.at[pl.ds(base, _RB)], xrows)
        pltpu.sync_copy(d1_hbm.at[pl.ds(base, _RB)], i1v)
        pltpu.sync_copy(d2_hbm.at[pl.ds(base, _RB)], i2v)
        c1 = pltpu.async_copy(xrows, xs_hbm.at[i1v], sem)
        c2 = pltpu.async_copy(xrows, xs_hbm.at[i2v], sem)
        c1.wait()
        c2.wait()

    return scatter_kernel


def _grouped_body(te_ref, used_ref, xs_ref, w1_ref, w2_ref, ys_ref,
                  w1c_ref, w2c_ref):
    i = pl.program_id(0)
    prev = te_ref[jnp.maximum(i - 1, 0)]
    changed = jnp.logical_or(i == 0, te_ref[i] != prev)

    @pl.when(changed)
    def _cvt():
        w1c_ref[...] = w1_ref[0].astype(jnp.bfloat16)
        w2c_ref[...] = w2_ref[0].astype(jnp.bfloat16)

    @pl.when(used_ref[i] != 0)
    def _compute():
        xb = xs_ref[...].astype(jnp.bfloat16)
        h = lax.dot_general(xb, w1c_ref[...], (((1,), (0,)), ((), ())),
                            preferred_element_type=jnp.float32)
        h = h * jax.nn.sigmoid(h)
        ys_ref[...] = lax.dot_general(h.astype(jnp.bfloat16), w2c_ref[...],
                                      (((1,), (0,)), ((), ())),
                                      preferred_element_type=jnp.float32)


def _grouped(xs, w1, w2, tile_expert, used):
    return pl.pallas_call(
        _grouped_body,
        grid_spec=pltpu.PrefetchScalarGridSpec(
            num_scalar_prefetch=2,
            grid=(NP_,),
            in_specs=[
                pl.BlockSpec((TM_, D_), lambda i, te, u: (i, 0)),
                pl.BlockSpec((1, D_, H_), lambda i, te, u: (te[i], 0, 0)),
                pl.BlockSpec((1, H_, D_), lambda i, te, u: (te[i], 0, 0)),
            ],
            out_specs=pl.BlockSpec((TM_, D_), lambda i, te, u: (i, 0)),
            scratch_shapes=[
                pltpu.VMEM((D_, H_), jnp.bfloat16),
                pltpu.VMEM((H_, D_), jnp.bfloat16),
            ],
        ),
        out_shape=jax.ShapeDtypeStruct((P_, D_), jnp.float32),
        compiler_params=pltpu.CompilerParams(
            dimension_semantics=("arbitrary",),
        ),
    )(tile_expert, used, xs, w1, w2)


def _make_gather():
    mesh = plsc.VectorSubcoreMesh(core_axis_name="core",
                                  subcore_axis_name="subcore")

    sb = _RB // 2

    @functools.partial(
        pl.kernel, mesh=mesh,
        out_type=[jax.ShapeDtypeStruct((T_, D_), jnp.float32),
                  jax.ShapeDtypeStruct((T_, D_), jnp.float32)],
        scratch_types=[pltpu.VMEM((sb, D_), jnp.float32),
                       pltpu.VMEM((sb, D_), jnp.float32),
                       pltpu.VMEM((sb,), jnp.int32),
                       pltpu.VMEM((sb,), jnp.int32),
                       pltpu.SemaphoreType.DMA])
    def gather_kernel(ys_hbm, d1_hbm, d2_hbm, g1_hbm, g2_hbm,
                      b1, b2, i1v, i2v, sem):
        wid = (lax.axis_index("subcore") * plsc.get_sparse_core_info().num_cores
               + lax.axis_index("core"))
        for s in range(2):
            base = wid * _RB + s * sb
            pltpu.sync_copy(d1_hbm.at[pl.ds(base, sb)], i1v)
            pltpu.sync_copy(d2_hbm.at[pl.ds(base, sb)], i2v)
            c1 = pltpu.async_copy(ys_hbm.at[i1v], b1, sem)
            c2 = pltpu.async_copy(ys_hbm.at[i2v], b2, sem)
            c1.wait()
            c2.wait()
            pltpu.sync_copy(b1, g1_hbm.at[pl.ds(base, sb)])
            pltpu.sync_copy(b2, g2_hbm.at[pl.ds(base, sb)])

    return gather_kernel


def _combine_body(g1_ref, g2_ref, wt1_ref, wt2_ref, out_ref):
    out_ref[...] = wt1_ref[...] * g1_ref[...] + wt2_ref[...] * g2_ref[...]


def _combine(g1, g2, wt1, wt2):
    nblk = T_ // CHUNK_
    return pl.pallas_call(
        _combine_body,
        grid=(nblk,),
        in_specs=[
            pl.BlockSpec((CHUNK_, D_), lambda i: (i, 0)),
            pl.BlockSpec((CHUNK_, D_), lambda i: (i, 0)),
            pl.BlockSpec((CHUNK_, 1), lambda i: (i, 0)),
            pl.BlockSpec((CHUNK_, 1), lambda i: (i, 0)),
        ],
        out_specs=pl.BlockSpec((CHUNK_, D_), lambda i: (i, 0)),
        out_shape=jax.ShapeDtypeStruct((T_, D_), jnp.float32),
    )(g1, g2, wt1, wt2)


@jax.jit
def kernel(x, w_router, w1, w2):
    d1, d2, wt1, wt2, counts = _router(x, w_router)

    c = counts[0]
    nt = (c + (TM_ - 1)) // TM_
    cumtiles = jnp.cumsum(nt)
    tidx = jnp.arange(NP_, dtype=jnp.int32)
    tile_expert = jnp.minimum(
        jnp.sum(tidx[:, None] >= cumtiles[None, :], axis=1).astype(jnp.int32),
        E_ - 1)
    seg_end = ((cumtiles - nt) * TM_ + c)[tile_expert]
    used = (tidx * TM_ < seg_end).astype(jnp.int32)

    d1r = d1.reshape(T_)
    d2r = d2.reshape(T_)

    xs = _make_scatter()(x, d1r, d2r)
    ys = _grouped(xs, w1, w2, tile_expert, used)
    g1, g2 = _make_gather()(ys, d1r, d2r)
    return _combine(g1, g2, wt1, wt2)
